# baseline (speedup 1.0000x reference)
import jax
import jax.numpy as jnp
from jax.experimental import pallas as pl
from jax.experimental.pallas import tpu as pltpu

_P = 68
_NROW = _P * _P
_PAD = 72
_NPADROW = _NROW + 2 * _PAD
_ZB = 14
_NS = 5
_NG = _ZB + 2
_KL = 16 * _NG
_OL = 16 * _ZB


def kernel(occ, w1, w2, fc_w, fc_b):
    B, D, H, W = occ.shape
    o = occ.astype(jnp.float32)
    opl = jnp.pad(o, ((0, 0), (2, 8), (2, 2), (2, 2))).reshape(B, D + 10, _NROW)
    opl = jnp.pad(opl, ((0, 0), (0, 0), (_PAD, _PAD)))
    idx = jnp.arange(_NS)[:, None] * _ZB + jnp.arange(_NG + 2)[None, :]
    o_win = opl[:, idx, :]
    o_win = jnp.moveaxis(o_win, 2, 3)
    o_win = jnp.broadcast_to(o_win[..., None],
                             (B, _NS, _NPADROW, _NG + 2, 16))
    o_win = o_win.reshape(B, _NS, _NPADROW, 16 * (_NG + 2))

    w2fc = (w2.reshape(432, 32) @ fc_w.T).reshape(3, 9, 16, 10)
    rearr = jnp.transpose(w2fc, (0, 2, 1, 3))
    cols = [jnp.pad(rearr, ((q, _ZB - 1 - q), (0, 0), (0, 0), (0, 6)))
            for q in range(_ZB)]
    b6 = jnp.stack(cols, axis=3)
    b6 = jnp.pad(b6, ((0, 0), (0, 0), (0, 0), (0, _NG - _ZB), (0, 0)))
    b6 = b6.reshape(_KL, 9 * _KL).astype(jnp.bfloat16)

    fcb6 = jnp.pad(fc_b, (0, 6))
    fcb6 = jnp.tile(fcb6, _NG).reshape(1, _KL)

    w1tile = jnp.tile(w1.reshape(27, 16), (1, _NG))

    def body(owin_ref, b6_ref, fcb6_ref, w1t_ref, out_ref,
             os1_ref, os2_ref, ysum_ref, ycat_ref, ua_ref, ub_ref):
        @pl.when(jnp.logical_and(pl.program_id(0) == 0,
                                 pl.program_id(1) == 0))
        def _init():
            ycat_ref[...] = jnp.zeros_like(ycat_ref)

        os1_ref[...] = owin_ref[0, 0, :, 16:16 + _KL]
        os2_ref[...] = owin_ref[0, 0, :, 32:32 + _KL]

        def _pass(reader, a):
            terms = None
            for bb in (-1, 0, 1):
                for cc in (-1, 0, 1):
                    tap = ((a + 1) * 3 + (bb + 1)) * 3 + (cc + 1)
                    sft = _P * bb + cc
                    t = reader(sft) * w1t_ref[tap, :][None, :]
                    terms = t if terms is None else terms + t
            return terms

        ysum_ref[_PAD:_PAD + _NROW, :] = _pass(
            lambda sft: owin_ref[0, 0, pl.ds(_PAD + sft, _NROW), 0:_KL], -1)
        ysum_ref[_PAD:_PAD + _NROW, :] += _pass(
            lambda sft: os1_ref[pl.ds(_PAD + sft, _NROW), :], 0)
        t2 = _pass(lambda sft: os2_ref[pl.ds(_PAD + sft, _NROW), :], 1)
        ycat_ref[_PAD:_PAD + _NROW, :] = (
            (ysum_ref[_PAD:_PAD + _NROW, :] + t2)
            * os1_ref[_PAD:_PAD + _NROW, :]).astype(jnp.bfloat16)

        acc = None
        for bb in (-1, 0, 1):
            for cc in (-1, 0, 1):
                g = (bb + 1) * 3 + (cc + 1)
                sft = _P * bb + cc
                u_ref = ua_ref if g % 2 == 0 else ub_ref
                u_ref[...] = jnp.dot(
                    ycat_ref[...], b6_ref[:, _KL * g:_KL * g + _KL],
                    preferred_element_type=jnp.float32)
                t = u_ref[pl.ds(_PAD + sft, _NROW), :]
                acc = t if acc is None else acc + t
        full = ((acc + fcb6_ref[0, :][None, :])
                * os2_ref[_PAD:_PAD + _NROW, :])
        out_ref[0, 0, :, :] = full[:, 0:_OL]

    out = pl.pallas_call(
        body,
        grid=(B, _NS),
        in_specs=[
            pl.BlockSpec((1, 1, _NPADROW, 16 * (_NG + 2)),
                         lambda b, st: (b, st, 0, 0)),
            pl.BlockSpec(b6.shape, lambda b, st: (0, 0)),
            pl.BlockSpec(fcb6.shape, lambda b, st: (0, 0)),
            pl.BlockSpec(w1tile.shape, lambda b, st: (0, 0)),
        ],
        out_specs=pl.BlockSpec((1, 1, _NROW, _OL),
                               lambda b, st: (b, st, 0, 0)),
        out_shape=jax.ShapeDtypeStruct((B, _NS, _NROW, _OL), jnp.float32),
        scratch_shapes=[
            pltpu.VMEM((_NPADROW, _KL), jnp.float32),
            pltpu.VMEM((_NPADROW, _KL), jnp.float32),
            pltpu.VMEM((_NPADROW, _KL), jnp.float32),
            pltpu.VMEM((_NPADROW, _KL), jnp.bfloat16),
            pltpu.VMEM((_NPADROW, _KL), jnp.float32),
            pltpu.VMEM((_NPADROW, _KL), jnp.float32),
        ],
    )(o_win, b6, fcb6, w1tile)

    out = out.reshape(B, _NS, _NROW, _ZB, 16)
    out = jnp.moveaxis(out, 3, 2).reshape(B, _NS * _ZB, _NROW, 16)
    out = out[:, :D, :, :10].reshape(B, D, _P, _P, 10)
    out = out[:, :, 2:2 + H, 2:2 + W, :]
    return out.reshape(B * D * H * W, 10)


# speedup vs baseline: 1.1001x; 1.1001x over previous
import jax
import jax.numpy as jnp
from jax.experimental import pallas as pl
from jax.experimental.pallas import tpu as pltpu

_P = 68
_NROW = _P * _P
_PAD = 72
_NPADROW = _NROW + 2 * _PAD
_ZB = 6
_NS = 11


def kernel(occ, w1, w2, fc_w, fc_b):
    B, D, H, W = occ.shape
    o = occ.astype(jnp.float32)
    opl = jnp.pad(o, ((0, 0), (2, 4), (2, 2), (2, 2))).reshape(B, D + 6, _NROW)
    opl = jnp.pad(opl, ((0, 0), (0, 0), (_PAD, _PAD)))
    idx = jnp.arange(_NS)[:, None] * _ZB + jnp.arange(10)[None, :]
    o_win = opl[:, idx, :]
    o_win = jnp.moveaxis(o_win, 2, 3)
    o_win = jnp.broadcast_to(o_win[..., None], (B, _NS, _NPADROW, 10, 16))
    o_win = o_win.reshape(B, _NS, _NPADROW, 160)

    w2fc = (w2.reshape(432, 32) @ fc_w.T).reshape(3, 9, 16, 10)
    rearr = jnp.transpose(w2fc, (0, 2, 1, 3))
    cols = [jnp.pad(rearr, ((q, 5 - q), (0, 0), (0, 0), (0, 6)))
            for q in range(_ZB)]
    b6 = jnp.stack(cols, axis=3)
    b6 = jnp.pad(b6, ((0, 0), (0, 0), (0, 0), (0, 2), (0, 0)))
    b6 = b6.reshape(128, 9 * 128).astype(jnp.bfloat16)

    fcb6 = jnp.pad(fc_b, (0, 6))
    fcb6 = jnp.tile(fcb6, 8).reshape(1, 128)

    w1r = w1.reshape(27, 16)
    w1tile = jnp.tile(w1r, (1, 8))

    def body(owin_ref, b6_ref, fcb6_ref, w1t_ref, out_ref,
             os1_ref, os2_ref, ysum_ref, ycat_ref, u_ref):
        @pl.when(jnp.logical_and(pl.program_id(0) == 0,
                                 pl.program_id(1) == 0))
        def _init():
            ycat_ref[...] = jnp.zeros_like(ycat_ref)

        os1_ref[...] = owin_ref[0, 0, :, 16:144]
        os2_ref[...] = owin_ref[0, 0, :, 32:160]

        def _pass(reader, a):
            terms = None
            for bb in (-1, 0, 1):
                for cc in (-1, 0, 1):
                    tap = ((a + 1) * 3 + (bb + 1)) * 3 + (cc + 1)
                    sft = _P * bb + cc
                    t = reader(sft) * w1t_ref[tap, :][None, :]
                    terms = t if terms is None else terms + t
            return terms

        ysum_ref[_PAD:_PAD + _NROW, :] = _pass(
            lambda sft: owin_ref[0, 0, pl.ds(_PAD + sft, _NROW), 0:128], -1)
        ysum_ref[_PAD:_PAD + _NROW, :] += _pass(
            lambda sft: os1_ref[pl.ds(_PAD + sft, _NROW), :], 0)
        t2 = _pass(lambda sft: os2_ref[pl.ds(_PAD + sft, _NROW), :], 1)
        ycat_ref[_PAD:_PAD + _NROW, :] = (
            (ysum_ref[_PAD:_PAD + _NROW, :] + t2)
            * os1_ref[_PAD:_PAD + _NROW, :]).astype(jnp.bfloat16)

        u_ref[...] = jnp.dot(ycat_ref[...], b6_ref[...],
                             preferred_element_type=jnp.float32)

        acc = None
        for bb in (-1, 0, 1):
            for cc in (-1, 0, 1):
                g = (bb + 1) * 3 + (cc + 1)
                sft = _P * bb + cc
                t = u_ref[pl.ds(_PAD + sft, _NROW), 128 * g:128 * g + 128]
                acc = t if acc is None else acc + t
        full = ((acc + fcb6_ref[0, :][None, :])
                * os2_ref[_PAD:_PAD + _NROW, :])
        out_ref[0, 0, :, :] = full[:, 0:96]

    out = pl.pallas_call(
        body,
        grid=(B, _NS),
        in_specs=[
            pl.BlockSpec((1, 1, _NPADROW, 160), lambda b, st: (b, st, 0, 0)),
            pl.BlockSpec(b6.shape, lambda b, st: (0, 0)),
            pl.BlockSpec(fcb6.shape, lambda b, st: (0, 0)),
            pl.BlockSpec(w1tile.shape, lambda b, st: (0, 0)),
        ],
        out_specs=pl.BlockSpec((1, 1, _NROW, 96), lambda b, st: (b, st, 0, 0)),
        out_shape=jax.ShapeDtypeStruct((B, _NS, _NROW, 96), jnp.float32),
        scratch_shapes=[
            pltpu.VMEM((_NPADROW, 128), jnp.float32),
            pltpu.VMEM((_NPADROW, 128), jnp.float32),
            pltpu.VMEM((_NPADROW, 128), jnp.float32),
            pltpu.VMEM((_NPADROW, 128), jnp.bfloat16),
            pltpu.VMEM((_NPADROW, 9 * 128), jnp.float32),
        ],
    )(o_win, b6, fcb6, w1tile)

    out = out.reshape(B, _NS, _NROW, _ZB, 16)
    out = jnp.moveaxis(out, 3, 2).reshape(B, _NS * _ZB, _NROW, 16)
    out = out[:, :D, :, :10].reshape(B, D, _P, _P, 10)
    out = out[:, :, 2:2 + H, 2:2 + W, :]
    return out.reshape(B * D * H * W, 10)
